# Initial kernel scaffold; baseline (speedup 1.0000x reference)
import jax
import jax.numpy as jnp
from jax.experimental import pallas as pl


def kernel(x, mask, weight, bias):
    raise NotImplementedError("write your pallas kernel here")


# speedup vs baseline: 9.7652x; 9.7652x over previous
import functools

import jax
import jax.numpy as jnp
from jax import lax
from jax.experimental import pallas as pl

_ZERO_TOL = 1e-06
_K = 3


def _wos_body(x_ref, m_ref, w_ref, b_ref, out_ref, *, rows, ow, c_in, d2):
    rc = pl.program_id(1)
    r0 = rc * rows

    pieces = []
    for ci in range(c_in):
        xc = x_ref[0, ci, pl.ds(r0, rows + _K - 1), :]
        for di in range(_K):
            for dj in range(_K):
                p = xc[di:di + rows, dj:dj + ow]
                pieces.append(p[None])
    em = jnp.concatenate(pieces, axis=0)
    em = jnp.concatenate([em, -em], axis=0)

    mcol = m_ref[...][0][:, :, None]
    v = em + mcol

    wcol = w_ref[...][0]
    nzc = wcol > _ZERO_TOL
    wm = jnp.where(nzc, wcol, 0.0)[:, :, None]
    bv = b_ref[...][0]

    big = jnp.float32(3.0e38)
    ymin = jnp.full((rows, ow), big, jnp.float32)
    ymax = jnp.full((rows, ow), -big, jnp.float32)
    found = jnp.zeros((rows, ow), jnp.bool_)

    for j in range(d2):
        vj = v[j]
        lo = jnp.where(v[:j + 1] >= vj[None], wm[:j + 1], 0.0)
        cj = jnp.sum(lo, axis=0)
        if j + 1 < d2:
            hi = jnp.where(v[j + 1:] > vj[None], wm[j + 1:], 0.0)
            cj = cj + jnp.sum(hi, axis=0)
        nzj = nzc[j:j + 1, :]
        ok = jnp.logical_and(cj <= bv, nzj)
        ymin = jnp.where(ok, jnp.minimum(ymin, vj), ymin)
        found = jnp.logical_or(found, ok)
        ymax = jnp.where(nzj, jnp.maximum(ymax, vj), ymax)

    y = jnp.where(found, ymin, jnp.where(ymax > -big, ymax, 0.0))
    out_ref[0] = y


def kernel(x, mask, weight, bias):
    B, C, H, W = x.shape
    NC, D2 = weight.shape
    oh, ow = H - _K + 1, W - _K + 1
    rows = 16
    RC = oh // rows
    N = B * oh * ow

    mask_t = mask.reshape(NC, D2, 1)
    weight_t = weight.reshape(NC, D2, 1)
    bias3 = bias.reshape(NC, 1, 1)

    body = functools.partial(_wos_body, rows=rows, ow=ow, c_in=C, d2=D2)
    out3 = pl.pallas_call(
        body,
        grid=(B, RC, NC),
        in_specs=[
            pl.BlockSpec((1, C, H, W), lambda b, rc, c: (b, 0, 0, 0)),
            pl.BlockSpec((1, D2, 1), lambda b, rc, c: (c, 0, 0)),
            pl.BlockSpec((1, D2, 1), lambda b, rc, c: (c, 0, 0)),
            pl.BlockSpec((1, 1, 1), lambda b, rc, c: (c, 0, 0)),
        ],
        out_specs=pl.BlockSpec(
            (1, rows, ow),
            lambda b, rc, c: (c, b * RC + rc, 0)),
        out_shape=jax.ShapeDtypeStruct((NC, B * oh, ow), jnp.float32),
    )(x, mask_t, weight_t, bias3)

    y = out3.reshape(NC, N).T
    return y.reshape(-1, NC, oh, ow)


# speedup vs baseline: 31.8908x; 3.2658x over previous
import functools

import jax
import jax.numpy as jnp
from jax import lax
from jax.experimental import pallas as pl

_ZERO_TOL = 1e-06
_K = 3


def _wos_body(x_ref, m_ref, w_ref, b_ref, out_ref, *, rows, ow, c_in, d2):
    rc = pl.program_id(1)
    r0 = rc * rows

    pieces = []
    for ci in range(c_in):
        xc = x_ref[0, ci, pl.ds(r0, rows + _K - 1), :]
        for di in range(_K):
            for dj in range(_K):
                p = xc[di:di + rows, dj:dj + ow]
                pieces.append(p[None])
    em = jnp.concatenate(pieces, axis=0)
    em = jnp.concatenate([em, -em], axis=0)
    em = jnp.concatenate([em, em], axis=2)

    def lane_pair(r):
        a = jnp.broadcast_to(r[0][:, :, None], (d2, 1, ow))
        b = jnp.broadcast_to(r[1][:, :, None], (d2, 1, ow))
        return jnp.concatenate([a, b], axis=2)

    mlane = lane_pair(m_ref[...])
    v = em + mlane

    wlane = lane_pair(w_ref[...])
    nzl = wlane > _ZERO_TOL
    wm = jnp.where(nzl, wlane, 0.0)

    br = b_ref[...]
    blane = jnp.concatenate(
        [jnp.broadcast_to(br[0], (1, ow)),
         jnp.broadcast_to(br[1], (1, ow))], axis=1)

    big = jnp.float32(3.0e38)
    ymin = jnp.full((rows, 2 * ow), big, jnp.float32)
    ymax = jnp.full((rows, 2 * ow), -big, jnp.float32)
    found = jnp.zeros((rows, 2 * ow), jnp.bool_)

    for j in range(d2):
        vj = v[j]
        lo = jnp.where(v[:j + 1] >= vj[None], wm[:j + 1], 0.0)
        cj = jnp.sum(lo, axis=0)
        if j + 1 < d2:
            hi = jnp.where(v[j + 1:] > vj[None], wm[j + 1:], 0.0)
            cj = cj + jnp.sum(hi, axis=0)
        nzj = nzl[j]
        ok = jnp.logical_and(cj <= blane, nzj)
        ymin = jnp.where(ok, jnp.minimum(ymin, vj), ymin)
        found = jnp.logical_or(found, ok)
        ymax = jnp.where(nzj, jnp.maximum(ymax, vj), ymax)

    y = jnp.where(found, ymin, jnp.where(ymax > -big, ymax, 0.0))
    out_ref[0] = y[:, :ow]
    out_ref[1] = y[:, ow:]


def kernel(x, mask, weight, bias):
    B, C, H, W = x.shape
    NC, D2 = weight.shape
    oh, ow = H - _K + 1, W - _K + 1
    rows = 16
    RC = oh // rows
    N = B * oh * ow

    mask3 = mask.reshape(NC, D2, 1)
    weight3 = weight.reshape(NC, D2, 1)
    bias3 = bias.reshape(NC, 1, 1)

    body = functools.partial(_wos_body, rows=rows, ow=ow, c_in=C, d2=D2)
    out3 = pl.pallas_call(
        body,
        grid=(B, RC, NC // 2),
        in_specs=[
            pl.BlockSpec((1, C, H, W), lambda b, rc, c: (b, 0, 0, 0)),
            pl.BlockSpec((2, D2, 1), lambda b, rc, c: (c, 0, 0)),
            pl.BlockSpec((2, D2, 1), lambda b, rc, c: (c, 0, 0)),
            pl.BlockSpec((2, 1, 1), lambda b, rc, c: (c, 0, 0)),
        ],
        out_specs=pl.BlockSpec(
            (2, rows, ow),
            lambda b, rc, c: (c, b * RC + rc, 0)),
        out_shape=jax.ShapeDtypeStruct((NC, B * oh, ow), jnp.float32),
    )(x, mask3, weight3, bias3)

    y = out3.reshape(NC, N).T
    return y.reshape(-1, NC, oh, ow)


# speedup vs baseline: 34.3809x; 1.0781x over previous
import functools

import jax
import jax.numpy as jnp
from jax import lax
from jax.experimental import pallas as pl

_ZERO_TOL = 1e-06
_K = 3


def _wos_body(x_ref, m_ref, w_ref, b_ref, out_ref, *, rows, ow, c_in, d2):
    rc = pl.program_id(1)
    r0 = rc * rows

    pieces = []
    for ci in range(c_in):
        xc = x_ref[0, ci, pl.ds(r0, rows + _K - 1), :]
        for di in range(_K):
            for dj in range(_K):
                p = xc[di:di + rows, dj:dj + ow]
                pieces.append(p[None])
    em = jnp.concatenate(pieces, axis=0)
    em = jnp.concatenate([em, -em], axis=0)
    em = jnp.concatenate([em, em], axis=2)

    def lane_pair(r):
        a = jnp.broadcast_to(r[0][:, :, None], (d2, 1, ow))
        b = jnp.broadcast_to(r[1][:, :, None], (d2, 1, ow))
        return jnp.concatenate([a, b], axis=2)

    mlane = lane_pair(m_ref[...])
    v = em + mlane

    wlane = lane_pair(w_ref[...])
    nzl = wlane > _ZERO_TOL
    wm = jnp.where(nzl, wlane, 0.0)

    br = b_ref[...]
    blane = jnp.concatenate(
        [jnp.broadcast_to(br[0], (1, ow)),
         jnp.broadcast_to(br[1], (1, ow))], axis=1)

    big = jnp.float32(3.0e38)
    ymin = jnp.full((rows, 2 * ow), big, jnp.float32)
    ymax = jnp.full((rows, 2 * ow), -big, jnp.float32)
    found = jnp.zeros((rows, 2 * ow), jnp.bool_)

    for j in range(d2):
        vj = v[j]
        lo = jnp.where(v[:j + 1] >= vj[None], wm[:j + 1], 0.0)
        cj = jnp.sum(lo, axis=0)
        if j + 1 < d2:
            hi = jnp.where(v[j + 1:] > vj[None], wm[j + 1:], 0.0)
            cj = cj + jnp.sum(hi, axis=0)
        nzj = nzl[j]
        ok = jnp.logical_and(cj <= blane, nzj)
        ymin = jnp.where(ok, jnp.minimum(ymin, vj), ymin)
        found = jnp.logical_or(found, ok)
        ymax = jnp.where(nzj, jnp.maximum(ymax, vj), ymax)

    y = jnp.where(found, ymin, jnp.where(ymax > -big, ymax, 0.0))
    out_ref[0] = y[:, :ow]
    out_ref[1] = y[:, ow:]


def kernel(x, mask, weight, bias):
    B, C, H, W = x.shape
    NC, D2 = weight.shape
    oh, ow = H - _K + 1, W - _K + 1
    rows = 32
    RC = oh // rows
    N = B * oh * ow

    mask3 = mask.reshape(NC, D2, 1)
    weight3 = weight.reshape(NC, D2, 1)
    bias3 = bias.reshape(NC, 1, 1)

    body = functools.partial(_wos_body, rows=rows, ow=ow, c_in=C, d2=D2)
    out3 = pl.pallas_call(
        body,
        grid=(B, RC, NC // 2),
        in_specs=[
            pl.BlockSpec((1, C, H, W), lambda b, rc, c: (b, 0, 0, 0)),
            pl.BlockSpec((2, D2, 1), lambda b, rc, c: (c, 0, 0)),
            pl.BlockSpec((2, D2, 1), lambda b, rc, c: (c, 0, 0)),
            pl.BlockSpec((2, 1, 1), lambda b, rc, c: (c, 0, 0)),
        ],
        out_specs=pl.BlockSpec(
            (2, rows, ow),
            lambda b, rc, c: (c, b * RC + rc, 0)),
        out_shape=jax.ShapeDtypeStruct((NC, B * oh, ow), jnp.float32),
    )(x, mask3, weight3, bias3)

    y = out3.reshape(NC, N).T
    return y.reshape(-1, NC, oh, ow)


# speedup vs baseline: 35.8870x; 1.0438x over previous
import functools

import jax
import jax.numpy as jnp
from jax import lax
from jax.experimental import pallas as pl

_ZERO_TOL = 1e-06
_K = 3


def _wos_body(x_ref, m_ref, w_ref, b_ref, out_ref, *, rows, ow, c_in, d2):
    rc = pl.program_id(1)
    r0 = rc * rows

    pieces = []
    for ci in range(c_in):
        xc = x_ref[0, ci, pl.ds(r0, rows + _K - 1), :]
        for di in range(_K):
            for dj in range(_K):
                p = xc[di:di + rows, dj:dj + ow]
                pieces.append(p[None])
    em = jnp.concatenate(pieces, axis=0)
    em = jnp.concatenate([em, -em], axis=0)
    em = jnp.concatenate([em, em], axis=2)

    def lane_pair(r):
        a = jnp.broadcast_to(r[0][:, :, None], (d2, 1, ow))
        b = jnp.broadcast_to(r[1][:, :, None], (d2, 1, ow))
        return jnp.concatenate([a, b], axis=2)

    mlane = lane_pair(m_ref[...])
    v = em + mlane

    wlane = lane_pair(w_ref[...])
    nzl = wlane > _ZERO_TOL
    wm = jnp.where(nzl, wlane, 0.0)

    br = b_ref[...]
    blane = jnp.concatenate(
        [jnp.broadcast_to(br[0], (1, ow)),
         jnp.broadcast_to(br[1], (1, ow))], axis=1)

    big = jnp.float32(3.0e38)
    ymin = jnp.full((rows, 2 * ow), big, jnp.float32)
    ymax = jnp.full((rows, 2 * ow), -big, jnp.float32)
    found = jnp.zeros((rows, 2 * ow), jnp.bool_)

    for j in range(d2):
        vj = v[j]
        lo = jnp.where(v[:j + 1] >= vj[None], wm[:j + 1], 0.0)
        cj = jnp.sum(lo, axis=0)
        if j + 1 < d2:
            hi = jnp.where(v[j + 1:] > vj[None], wm[j + 1:], 0.0)
            cj = cj + jnp.sum(hi, axis=0)
        nzj = nzl[j]
        ok = jnp.logical_and(cj <= blane, nzj)
        ymin = jnp.where(ok, jnp.minimum(ymin, vj), ymin)
        found = jnp.logical_or(found, ok)
        ymax = jnp.where(nzj, jnp.maximum(ymax, vj), ymax)

    y = jnp.where(found, ymin, jnp.where(ymax > -big, ymax, 0.0))
    out_ref[0] = y[:, :ow]
    out_ref[1] = y[:, ow:]


def kernel(x, mask, weight, bias):
    B, C, H, W = x.shape
    NC, D2 = weight.shape
    oh, ow = H - _K + 1, W - _K + 1
    rows = 64
    RC = oh // rows
    N = B * oh * ow

    mask3 = mask.reshape(NC, D2, 1)
    weight3 = weight.reshape(NC, D2, 1)
    bias3 = bias.reshape(NC, 1, 1)

    body = functools.partial(_wos_body, rows=rows, ow=ow, c_in=C, d2=D2)
    out3 = pl.pallas_call(
        body,
        grid=(B, RC, NC // 2),
        in_specs=[
            pl.BlockSpec((1, C, H, W), lambda b, rc, c: (b, 0, 0, 0)),
            pl.BlockSpec((2, D2, 1), lambda b, rc, c: (c, 0, 0)),
            pl.BlockSpec((2, D2, 1), lambda b, rc, c: (c, 0, 0)),
            pl.BlockSpec((2, 1, 1), lambda b, rc, c: (c, 0, 0)),
        ],
        out_specs=pl.BlockSpec(
            (2, rows, ow),
            lambda b, rc, c: (c, b * RC + rc, 0)),
        out_shape=jax.ShapeDtypeStruct((NC, B * oh, ow), jnp.float32),
    )(x, mask3, weight3, bias3)

    y = out3.reshape(NC, N).T
    return y.reshape(-1, NC, oh, ow)
